# baseline (speedup 1.0000x reference)
import functools

import jax
import jax.numpy as jnp
from jax import lax
from jax.experimental import pallas as pl
from jax.experimental.pallas import tpu as pltpu
from jax.experimental.pallas import tpu_sc as plsc

_DIM = 768
_NC = 2
_NS = 16
_NW = _NC * _NS
_CHUNK = 32
_NBUF = 4


def _make_gather(n_total: int, dim: int):
    steps = n_total // (_NW * _CHUNK)
    mesh = plsc.VectorSubcoreMesh(core_axis_name="c", subcore_axis_name="s")

    @functools.partial(
        pl.kernel,
        mesh=mesh,
        out_type=jax.ShapeDtypeStruct((n_total, dim), jnp.float32),
        scratch_types=[
            pltpu.VMEM((steps, _CHUNK), jnp.int32),
            pltpu.VMEM((steps, _CHUNK), jnp.int32),
            pltpu.VMEM((_NBUF, _CHUNK, dim), jnp.float32),
            pltpu.SemaphoreType.DMA((_NBUF,)),
            pltpu.SemaphoreType.DMA((_NBUF,)),
        ],
    )
    def k(table_hbm, idx_hbm, out_hbm, idx_v, oidx_v, bufs, gsem, osem):
        wid = lax.axis_index("s") * _NC + lax.axis_index("c")
        base = wid * (steps * _CHUNK)
        pltpu.sync_copy(idx_hbm.at[wid], idx_v)

        lane = lax.iota(jnp.int32, 16)
        for j in range(steps):
            for kk in range(_CHUNK // 16):
                oidx_v[j, pl.ds(kk * 16, 16)] = base + j * _CHUNK + kk * 16 + lane

        gathers = [None] * steps
        out_cp = [None] * _NBUF
        for j in range(min(_NBUF - 1, steps)):
            gathers[j] = pltpu.async_copy(
                table_hbm.at[idx_v.at[j]], bufs.at[j], gsem.at[j])
        for j in range(steps):
            b = j % _NBUF
            gathers[j].wait()
            jn = j + _NBUF - 1
            if jn < steps:
                nb = jn % _NBUF
                if out_cp[nb] is not None:
                    out_cp[nb].wait()
                    out_cp[nb] = None
                gathers[jn] = pltpu.async_copy(
                    table_hbm.at[idx_v.at[jn]], bufs.at[nb], gsem.at[nb])
            out_cp[b] = pltpu.async_copy(
                bufs.at[b], out_hbm.at[oidx_v.at[j]], osem.at[b])
        for b in range(_NBUF):
            if out_cp[b] is not None:
                out_cp[b].wait()

    return k


def kernel(position_ids, table):
    n_total = position_ids.size
    idx = position_ids.astype(jnp.int32).reshape(
        _NW, n_total // (_NW * _CHUNK), _CHUNK)
    out = _make_gather(n_total, table.shape[1])(table, idx)
    return out.reshape(position_ids.shape + (table.shape[1],))


# speedup vs baseline: 1.0324x; 1.0324x over previous
import functools

import jax
import jax.numpy as jnp
from jax import lax
from jax.experimental import pallas as pl
from jax.experimental.pallas import tpu as pltpu
from jax.experimental.pallas import tpu_sc as plsc

_DIM = 768
_NC = 2
_NS = 16
_NW = _NC * _NS
_CHUNK = 32
_NBUF = 5


def _make_gather(n_total: int, dim: int):
    steps = n_total // (_NW * _CHUNK)
    mesh = plsc.VectorSubcoreMesh(core_axis_name="c", subcore_axis_name="s")

    @functools.partial(
        pl.kernel,
        mesh=mesh,
        out_type=jax.ShapeDtypeStruct((n_total, dim), jnp.float32),
        scratch_types=[
            pltpu.VMEM((steps, _CHUNK), jnp.int32),
            pltpu.VMEM((_NBUF, _CHUNK, dim), jnp.float32),
            pltpu.SemaphoreType.DMA((_NBUF,)),
            pltpu.SemaphoreType.DMA((_NBUF,)),
        ],
    )
    def k(table_hbm, idx_hbm, out_hbm, idx_v, bufs, gsem, osem):
        wid = lax.axis_index("s") * _NC + lax.axis_index("c")
        base = wid * (steps * _CHUNK)
        pltpu.sync_copy(idx_hbm.at[wid], idx_v)

        gathers = [None] * steps
        out_cp = [None] * _NBUF
        for j in range(min(_NBUF - 1, steps)):
            gathers[j] = pltpu.async_copy(
                table_hbm.at[idx_v.at[j]], bufs.at[j], gsem.at[j])
        for j in range(steps):
            b = j % _NBUF
            gathers[j].wait()
            out_cp[b] = pltpu.async_copy(
                bufs.at[b], out_hbm.at[pl.ds(base + j * _CHUNK, _CHUNK)], osem.at[b])
            jn = j + _NBUF - 1
            if jn < steps:
                nb = jn % _NBUF
                if out_cp[nb] is not None:
                    out_cp[nb].wait()
                    out_cp[nb] = None
                gathers[jn] = pltpu.async_copy(
                    table_hbm.at[idx_v.at[jn]], bufs.at[nb], gsem.at[nb])
        for b in range(_NBUF):
            if out_cp[b] is not None:
                out_cp[b].wait()

    return k


def kernel(position_ids, table):
    n_total = position_ids.size
    idx = position_ids.astype(jnp.int32).reshape(
        _NW, n_total // (_NW * _CHUNK), _CHUNK)
    out = _make_gather(n_total, table.shape[1])(table, idx)
    return out.reshape(position_ids.shape + (table.shape[1],))


# speedup vs baseline: 1.0393x; 1.0067x over previous
import functools

import jax
import jax.numpy as jnp
from jax import lax
from jax.experimental import pallas as pl
from jax.experimental.pallas import tpu as pltpu
from jax.experimental.pallas import tpu_sc as plsc

_DIM = 768
_NC = 2
_NS = 16
_NW = _NC * _NS
_CHUNK = 32
_NBUF = 5


def _make_gather(n_total: int, dim: int):
    steps = n_total // (_NW * _CHUNK)
    mesh = plsc.VectorSubcoreMesh(core_axis_name="c", subcore_axis_name="s")

    @functools.partial(
        pl.kernel,
        mesh=mesh,
        out_type=jax.ShapeDtypeStruct((n_total, dim), jnp.float32),
        scratch_types=[
            pltpu.VMEM((steps * _CHUNK,), jnp.int32),
            pltpu.VMEM((_NBUF, _CHUNK, dim), jnp.float32),
            pltpu.SemaphoreType.DMA((_NBUF,)),
            pltpu.SemaphoreType.DMA((_NBUF,)),
        ],
    )
    def k(table_hbm, idx_hbm, out_hbm, idx_v, bufs, gsem, osem):
        wid = lax.axis_index("s") * _NC + lax.axis_index("c")
        per_w = steps * _CHUNK
        base = wid * per_w
        seq = idx_hbm.shape[1]
        pltpu.sync_copy(
            idx_hbm.at[base // seq, pl.ds(base % seq, per_w)], idx_v)

        gathers = [None] * steps
        out_cp = [None] * _NBUF
        for j in range(min(_NBUF - 1, steps)):
            gathers[j] = pltpu.async_copy(
                table_hbm.at[idx_v.at[pl.ds(j * _CHUNK, _CHUNK)]], bufs.at[j], gsem.at[j])
        for j in range(steps):
            b = j % _NBUF
            gathers[j].wait()
            out_cp[b] = pltpu.async_copy(
                bufs.at[b], out_hbm.at[pl.ds(base + j * _CHUNK, _CHUNK)], osem.at[b])
            jn = j + _NBUF - 1
            if jn < steps:
                nb = jn % _NBUF
                if out_cp[nb] is not None:
                    out_cp[nb].wait()
                    out_cp[nb] = None
                gathers[jn] = pltpu.async_copy(
                    table_hbm.at[idx_v.at[pl.ds(jn * _CHUNK, _CHUNK)]], bufs.at[nb], gsem.at[nb])
        for b in range(_NBUF):
            if out_cp[b] is not None:
                out_cp[b].wait()

    return k


def kernel(position_ids, table):
    n_total = position_ids.size
    idx = position_ids.astype(jnp.int32)
    out = _make_gather(n_total, table.shape[1])(table, idx)
    return out.reshape(position_ids.shape + (table.shape[1],))
